# Initial kernel scaffold; baseline (speedup 1.0000x reference)
import jax
import jax.numpy as jnp
from jax.experimental import pallas as pl


def kernel(x, edge_index, W1, b1, W2, b2):
    raise NotImplementedError("write your pallas kernel here")


# speedup vs baseline: 43.1434x; 43.1434x over previous
import functools

import jax
import jax.numpy as jnp
from jax import lax
from jax.experimental import pallas as pl
from jax.experimental.pallas import tpu as pltpu
from jax.experimental.pallas import tpu_sc as plsc

NC = 2
NS = 16
NW = NC * NS
LANE = 16
GRP = 128
GB = 8
ROWBLK = 2048


def _cdiv(a, b):
    return (a + b - 1) // b


@functools.lru_cache(maxsize=None)
def _make_agg(npad, g_per_tile, d, gather):
    mesh = plsc.VectorSubcoreMesh(core_axis_name="c", subcore_axis_name="s",
                                  num_cores=NC, num_subcores=NS)
    rows_per_tile = npad // NS
    n_chunks = g_per_tile // GB
    vec2 = d > 1
    acc_shape = (npad, d) if vec2 else (npad,)
    out_shape = (NC, npad, d) if vec2 else (NC, npad)
    rows_shape = ((GB, GRP, d) if vec2 else (GB, GRP)) if gather else (GRP,)

    scratch = [
        pltpu.VMEM((GB, GRP), jnp.int32),
        pltpu.VMEM((GB, GRP), jnp.int32),
        pltpu.VMEM(rows_shape, jnp.float32),
        pltpu.VMEM_SHARED(acc_shape, jnp.float32),
        pltpu.SemaphoreType.DMA,
    ]

    def body(table_hbm, src_hbm, dst_hbm, zero_hbm, out_hbm,
             sidx, didx, rows, acc, sem):
        cid = lax.axis_index("c")
        sid = lax.axis_index("s")
        wid = sid * NC + cid
        row0 = sid * rows_per_tile
        pltpu.sync_copy(zero_hbm.at[pl.ds(row0, rows_per_tile)],
                        acc.at[pl.ds(row0, rows_per_tile)])
        if not gather:
            for j in range(GRP // LANE):
                rows[pl.ds(j * LANE, LANE)] = jnp.ones((LANE,), jnp.float32)
        plsc.subcore_barrier()

        g0 = wid * g_per_tile

        def chunk(i, carry):
            base = g0 + i * GB
            pltpu.sync_copy(dst_hbm.at[pl.ds(base, GB)], didx)
            if gather:
                pltpu.sync_copy(src_hbm.at[pl.ds(base, GB)], sidx)
                for g in range(GB):
                    pltpu.async_copy(table_hbm.at[sidx.at[g]], rows.at[g],
                                     sem).wait()
                    pltpu.sync_copy(rows.at[g], acc.at[didx.at[g]], add=True)
            else:
                for g in range(GB):
                    pltpu.sync_copy(rows, acc.at[didx.at[g]], add=True)
            return carry

        lax.fori_loop(0, n_chunks, chunk, 0)
        plsc.subcore_barrier()
        pltpu.sync_copy(acc.at[pl.ds(row0, rows_per_tile)],
                        out_hbm.at[cid, pl.ds(row0, rows_per_tile)])

    return pl.kernel(body,
                     out_type=jax.ShapeDtypeStruct(out_shape, jnp.float32),
                     mesh=mesh, scratch_types=scratch,
                     compiler_params=pltpu.CompilerParams(
                         use_tc_tiling_on_sc=False))


def _stage_deg(npad, f_in):
    grid = npad // ROWBLK

    def body(dp_ref, x_ref, u_ref, dinv_ref):
        deg = 1.0 + dp_ref[0, :] + dp_ref[1, :]
        dinv = lax.rsqrt(deg)[:, None]
        dinv_ref[...] = dinv
        u_ref[...] = x_ref[...] * dinv

    return pl.pallas_call(
        body,
        grid=(grid,),
        in_specs=[pl.BlockSpec((2, ROWBLK), lambda i: (0, i)),
                  pl.BlockSpec((ROWBLK, f_in), lambda i: (i, 0))],
        out_specs=[pl.BlockSpec((ROWBLK, f_in), lambda i: (i, 0)),
                   pl.BlockSpec((ROWBLK, 1), lambda i: (i, 0))],
        out_shape=[jax.ShapeDtypeStruct((npad, f_in), jnp.float32),
                   jax.ShapeDtypeStruct((npad, 1), jnp.float32)],
    )


def _stage_mid(npad, f_in, f_hid):
    grid = npad // ROWBLK

    def body(ap_ref, u_ref, dinv_ref, w1_ref, b1_ref, w2_ref, v_ref):
        a = ap_ref[0] + ap_ref[1] + u_ref[...]
        dinv = dinv_ref[...]
        t = jnp.dot(a, w1_ref[...], preferred_element_type=jnp.float32)
        h = jnp.maximum(t * dinv + b1_ref[...], 0.0)
        v_ref[...] = jnp.dot(h, w2_ref[...],
                             preferred_element_type=jnp.float32) * dinv

    return pl.pallas_call(
        body,
        grid=(grid,),
        in_specs=[pl.BlockSpec((2, ROWBLK, f_in), lambda i: (0, i, 0)),
                  pl.BlockSpec((ROWBLK, f_in), lambda i: (i, 0)),
                  pl.BlockSpec((ROWBLK, 1), lambda i: (i, 0)),
                  pl.BlockSpec((f_in, f_hid), lambda i: (0, 0)),
                  pl.BlockSpec((1, f_hid), lambda i: (0, 0)),
                  pl.BlockSpec((f_hid, 1), lambda i: (0, 0))],
        out_specs=pl.BlockSpec((ROWBLK, 1), lambda i: (i, 0)),
        out_shape=jax.ShapeDtypeStruct((npad, 1), jnp.float32),
    )


def _stage_out(npad):
    grid = npad // ROWBLK

    def body(ap_ref, v_ref, dinv_ref, b2_ref, o_ref):
        a = (ap_ref[0, :] + ap_ref[1, :])[:, None] + v_ref[...]
        o_ref[...] = a * dinv_ref[...] + b2_ref[...]

    return pl.pallas_call(
        body,
        grid=(grid,),
        in_specs=[pl.BlockSpec((2, ROWBLK), lambda i: (0, i)),
                  pl.BlockSpec((ROWBLK, 1), lambda i: (i, 0)),
                  pl.BlockSpec((ROWBLK, 1), lambda i: (i, 0)),
                  pl.BlockSpec((1, 1), lambda i: (0, 0))],
        out_specs=pl.BlockSpec((ROWBLK, 1), lambda i: (i, 0)),
        out_shape=jax.ShapeDtypeStruct((npad, 1), jnp.float32),
    )


def kernel(x, edge_index, W1, b1, W2, b2):
    n, f_in = x.shape
    f_hid = W1.shape[1]
    e = edge_index.shape[1]

    npad = _cdiv(n, ROWBLK) * ROWBLK
    if npad == n:
        npad += ROWBLK
    g_per_tile = _cdiv(e, GRP * NW * GB) * GB
    epad = g_per_tile * NW * GRP

    xp = jnp.zeros((npad, f_in), jnp.float32).at[:n].set(x)
    srcp = jnp.concatenate(
        [edge_index[0], jnp.full((epad - e,), n, jnp.int32)]).reshape(-1, GRP)
    dstp = jnp.concatenate(
        [edge_index[1], jnp.full((epad - e,), n, jnp.int32)]).reshape(-1, GRP)
    z1 = jnp.zeros((npad,), jnp.float32)
    z5 = jnp.zeros((npad, f_in), jnp.float32)

    deg_p = _make_agg(npad, g_per_tile, 1, False)(z1, srcp, dstp, z1)
    u, dinv = _stage_deg(npad, f_in)(deg_p, xp)
    agg1_p = _make_agg(npad, g_per_tile, f_in, True)(u, srcp, dstp, z5)
    v = _stage_mid(npad, f_in, f_hid)(agg1_p, u, dinv, W1,
                                      b1.reshape(1, f_hid), W2)
    agg2_p = _make_agg(npad, g_per_tile, 1, True)(v[:, 0], srcp, dstp, z1)
    out = _stage_out(npad)(agg2_p, v, dinv, b2.reshape(1, 1))
    return out[:n]


# speedup vs baseline: 121.1720x; 2.8086x over previous
import functools

import jax
import jax.numpy as jnp
from jax import lax
from jax.experimental import pallas as pl
from jax.experimental.pallas import tpu as pltpu
from jax.experimental.pallas import tpu_sc as plsc

NC = 2
NS = 16
NW = NC * NS
LANE = 16
GRP = 128
GB = 8
CHW = GB * GRP
ROWBLK = 2048


def _cdiv(a, b):
    return (a + b - 1) // b


@functools.lru_cache(maxsize=None)
def _make_agg(npad, n_chunks, d, mode):
    mesh = plsc.VectorSubcoreMesh(core_axis_name="c", subcore_axis_name="s",
                                  num_cores=NC, num_subcores=NS)
    rows_per_tile = npad // NS
    vec2 = d > 1
    acc_shape = (npad, d) if vec2 else (npad,)
    out_shape = (NC, npad, d) if vec2 else (NC, npad)
    rows_shape = ((GB, GRP, d) if vec2 else (GB, GRP)) if mode != "count" \
        else (GRP,)

    scratch = [
        pltpu.VMEM((GB, GRP) if mode == "dma_r1" else (2, GB, GRP),
                   jnp.int32),
        pltpu.VMEM((2, GB, GRP), jnp.int32),
        pltpu.VMEM(rows_shape, jnp.float32),
        pltpu.VMEM_SHARED(acc_shape, jnp.float32),
        pltpu.SemaphoreType.DMA,
        pltpu.SemaphoreType.DMA,
    ]
    if mode.startswith("dma"):
        scratch.extend([pltpu.SemaphoreType.DMA] * GB)
    if mode == "reg":
        scratch.append(pltpu.VMEM((npad,), jnp.float32))

    def body(table_hbm, src_hbm, dst_hbm, zero_hbm, out_hbm,
             sidx, didx, rows, acc, sem_i, sem_s, *rest):
        cid = lax.axis_index("c")
        sid = lax.axis_index("s")
        wid = sid * NC + cid
        row0 = sid * rows_per_tile
        pltpu.sync_copy(zero_hbm.at[pl.ds(row0, rows_per_tile)],
                        acc.at[pl.ds(row0, rows_per_tile)])
        if mode == "count":
            for j in range(GRP // LANE):
                rows[pl.ds(j * LANE, LANE)] = jnp.ones((LANE,), jnp.float32)
        if mode == "reg":
            vtab = rest[0]
            pltpu.sync_copy(table_hbm, vtab)
        plsc.subcore_barrier()

        nf = (n_chunks + NW - 1 - wid) // NW

        sem_g = rest[:GB] if mode.startswith("dma") else ()

        def idx_start(i, buf):
            base = (wid + i * NW) * GB
            if mode != "count":
                pltpu.async_copy(src_hbm.at[pl.ds(base, GB)],
                                 sidx.at[buf], sem_i)
            pltpu.async_copy(dst_hbm.at[pl.ds(base, GB)], didx.at[buf], sem_i)

        def idx_wait(i, buf):
            base = (wid + i * NW) * GB
            if mode != "count":
                pltpu.make_async_copy(src_hbm.at[pl.ds(base, GB)],
                                      sidx.at[buf],
                                      sem_i).wait()
            pltpu.make_async_copy(dst_hbm.at[pl.ds(base, GB)], didx.at[buf],
                                  sem_i).wait()

        if mode == "dma_r1":
            def chunk_r1(i, carry):
                base = (wid + i * NW) * GB
                pltpu.sync_copy(src_hbm.at[pl.ds(base, GB)], sidx)
                pltpu.sync_copy(dst_hbm.at[pl.ds(base, GB)], didx.at[0])
                for g in range(GB):
                    pltpu.async_copy(table_hbm.at[sidx.at[g]], rows.at[g],
                                     sem_g[0]).wait()
                    pltpu.sync_copy(rows.at[g], acc.at[didx.at[0, g]],
                                    add=True)
                return carry
            lax.fori_loop(0, nf, chunk_r1, 0)
            plsc.subcore_barrier()
            pltpu.sync_copy(acc.at[pl.ds(row0, rows_per_tile)],
                            out_hbm.at[cid, pl.ds(row0, rows_per_tile)])
            return

        @pl.when(nf > 0)
        def _():
            idx_start(0, 0)

        def process(i, cur):
            idx_wait(i, cur)

            @pl.when(i + 1 < nf)
            def _():
                idx_start(i + 1, 1 - cur)

            if mode == "dma":
                for g in range(GB):
                    pltpu.async_copy(
                        table_hbm.at[sidx.at[cur, g]],
                        rows.at[g], sem_g[g])
                for g in range(GB):
                    pltpu.make_async_copy(
                        table_hbm.at[sidx.at[cur, g]],
                        rows.at[g], sem_g[g]).wait()
                    pltpu.async_copy(rows.at[g], acc.at[didx.at[cur, g]],
                                     sem_s, add=True)
            elif mode == "dma_sync":
                for g in range(GB):
                    pltpu.async_copy(
                        table_hbm.at[sidx.at[cur, g]],
                        rows.at[g], sem_g[0]).wait()
                    pltpu.async_copy(rows.at[g], acc.at[didx.at[cur, g]],
                                     sem_s, add=True)
            elif mode == "reg":
                vtab = rest[0]
                lanes = lax.iota(jnp.int32, LANE)
                for g in range(GB):
                    gcur = jnp.full((LANE,), cur, jnp.int32)
                    grow = jnp.full((LANE,), g, jnp.int32)
                    for j in range(GRP // LANE):
                        idxv = plsc.load_gather(sidx, [gcur, grow,
                                                       j * LANE + lanes])
                        rows[g, pl.ds(j * LANE, LANE)] = \
                            plsc.load_gather(vtab, [idxv])
                    pltpu.async_copy(rows.at[g], acc.at[didx.at[cur, g]],
                                     sem_s, add=True)
            else:
                for g in range(GB):
                    pltpu.async_copy(rows, acc.at[didx.at[cur, g]],
                                     sem_s, add=True)
            for g in range(GB):
                src_ref = rows if mode == "count" else rows.at[g]
                pltpu.make_async_copy(src_ref, acc.at[didx.at[cur, g]],
                                      sem_s).wait()

        def pair(p, carry):
            process(2 * p, 0)
            process(2 * p + 1, 1)
            return carry

        lax.fori_loop(0, nf // 2, pair, 0)

        @pl.when(lax.rem(nf, 2) == 1)
        def _():
            process(nf - 1, 0)
        plsc.subcore_barrier()
        pltpu.sync_copy(acc.at[pl.ds(row0, rows_per_tile)],
                        out_hbm.at[cid, pl.ds(row0, rows_per_tile)])

    return pl.kernel(body,
                     out_type=jax.ShapeDtypeStruct(out_shape, jnp.float32),
                     mesh=mesh, scratch_types=scratch,
                     compiler_params=pltpu.CompilerParams(
                         use_tc_tiling_on_sc=False,
                         needs_layout_passes=(mode != "reg")))


def _stage_deg(npad, f_in):
    grid = npad // ROWBLK

    def body(dp_ref, x_ref, u_ref, dinv_ref):
        deg = 1.0 + dp_ref[0, :] + dp_ref[1, :]
        dinv = lax.rsqrt(deg)[:, None]
        dinv_ref[...] = dinv
        u_ref[...] = x_ref[...] * dinv

    return pl.pallas_call(
        body,
        grid=(grid,),
        in_specs=[pl.BlockSpec((2, ROWBLK), lambda i: (0, i)),
                  pl.BlockSpec((ROWBLK, f_in), lambda i: (i, 0))],
        out_specs=[pl.BlockSpec((ROWBLK, f_in), lambda i: (i, 0)),
                   pl.BlockSpec((ROWBLK, 1), lambda i: (i, 0))],
        out_shape=[jax.ShapeDtypeStruct((npad, f_in), jnp.float32),
                   jax.ShapeDtypeStruct((npad, 1), jnp.float32)],
    )


def _stage_mid(npad, f_in, f_hid):
    grid = npad // ROWBLK

    def body(ap_ref, u_ref, dinv_ref, w1_ref, b1_ref, w2_ref, v_ref):
        a = ap_ref[0] + ap_ref[1] + u_ref[...]
        dinv = dinv_ref[...]
        t = jnp.dot(a, w1_ref[...], preferred_element_type=jnp.float32)
        h = jnp.maximum(t * dinv + b1_ref[...], 0.0)
        v_ref[...] = jnp.dot(h, w2_ref[...],
                             preferred_element_type=jnp.float32) * dinv

    return pl.pallas_call(
        body,
        grid=(grid,),
        in_specs=[pl.BlockSpec((2, ROWBLK, f_in), lambda i: (0, i, 0)),
                  pl.BlockSpec((ROWBLK, f_in), lambda i: (i, 0)),
                  pl.BlockSpec((ROWBLK, 1), lambda i: (i, 0)),
                  pl.BlockSpec((f_in, f_hid), lambda i: (0, 0)),
                  pl.BlockSpec((1, f_hid), lambda i: (0, 0)),
                  pl.BlockSpec((f_hid, 1), lambda i: (0, 0))],
        out_specs=pl.BlockSpec((ROWBLK, 1), lambda i: (i, 0)),
        out_shape=jax.ShapeDtypeStruct((npad, 1), jnp.float32),
    )


def _stage_out(npad):
    grid = npad // ROWBLK

    def body(ap_ref, v_ref, dinv_ref, b2_ref, o_ref):
        a = (ap_ref[0, :] + ap_ref[1, :])[:, None] + v_ref[...]
        o_ref[...] = a * dinv_ref[...] + b2_ref[...]

    return pl.pallas_call(
        body,
        grid=(grid,),
        in_specs=[pl.BlockSpec((2, ROWBLK), lambda i: (0, i)),
                  pl.BlockSpec((ROWBLK, 1), lambda i: (i, 0)),
                  pl.BlockSpec((ROWBLK, 1), lambda i: (i, 0)),
                  pl.BlockSpec((1, 1), lambda i: (0, 0))],
        out_specs=pl.BlockSpec((ROWBLK, 1), lambda i: (i, 0)),
        out_shape=jax.ShapeDtypeStruct((npad, 1), jnp.float32),
    )


def kernel(x, edge_index, W1, b1, W2, b2):
    n, f_in = x.shape
    f_hid = W1.shape[1]
    e = edge_index.shape[1]

    npad = _cdiv(n, ROWBLK) * ROWBLK
    if npad == n:
        npad += ROWBLK

    if e % CHW == 0:
        src = edge_index[0]
        dst = edge_index[1]
        epad = e
    else:
        epad = _cdiv(e, CHW) * CHW
        fill = jnp.full((epad - e,), n, jnp.int32)
        src = jnp.concatenate([edge_index[0], fill])
        dst = jnp.concatenate([edge_index[1], fill])
    n_chunks = epad // CHW
    src2 = src.reshape(-1, GRP)
    dst2 = dst.reshape(-1, GRP)

    fp = 8
    xp = jnp.zeros((npad, fp), jnp.float32).at[:n, :f_in].set(x)
    w1p = jnp.zeros((fp, f_hid), jnp.float32).at[:f_in].set(W1)
    z1 = jnp.zeros((npad,), jnp.float32)
    z8 = jnp.zeros((npad, fp), jnp.float32)

    deg_p = _make_agg(npad, n_chunks, 1, "count")(z1, src2, dst2, z1)
    u, dinv = _stage_deg(npad, fp)(deg_p, xp)
    agg1_p = _make_agg(npad, n_chunks, fp, "dma")(u, src2, dst2, z8)
    v = _stage_mid(npad, fp, f_hid)(agg1_p, u, dinv, w1p,
                                    b1.reshape(1, f_hid), W2)
    agg2_p = _make_agg(npad, n_chunks, 1, "reg")(v[:, 0], src2, dst2, z1)
    out = _stage_out(npad)(agg2_p, v, dinv, b2.reshape(1, 1))
    return out[:n]


# speedup vs baseline: 155.3662x; 1.2822x over previous
import functools

import jax
import jax.numpy as jnp
from jax import lax
from jax.experimental import pallas as pl
from jax.experimental.pallas import tpu as pltpu
from jax.experimental.pallas import tpu_sc as plsc

NC = 2
NS = 16
NW = NC * NS
LANE = 16
GRP = 128
GB = 8
CHW = GB * GRP
ROWBLK = 2048


def _cdiv(a, b):
    return (a + b - 1) // b


@functools.lru_cache(maxsize=None)
def _make_agg(npad, n_chunks, d, mode):
    mesh = plsc.VectorSubcoreMesh(core_axis_name="c", subcore_axis_name="s",
                                  num_cores=NC, num_subcores=NS)
    rows_per_tile = npad // NS
    vec2 = d > 1
    acc_shape = (npad, d) if vec2 else (npad,)
    out_shape = (NC, npad, d) if vec2 else (NC, npad)
    rows_shape = ((GB, GRP, d) if vec2 else (GB, GRP)) if mode != "count" \
        else (GRP,)

    scratch = [
        pltpu.VMEM((GB, GRP) if mode == "dma_r1" else (2, GB, GRP),
                   jnp.int32),
        pltpu.VMEM((2, GB, GRP), jnp.int32),
        pltpu.VMEM(rows_shape, jnp.float32),
        pltpu.VMEM_SHARED(acc_shape, jnp.float32),
        pltpu.SemaphoreType.DMA,
        pltpu.SemaphoreType.DMA,
    ]
    if mode.startswith("dma"):
        scratch.extend([pltpu.SemaphoreType.DMA] * GB)
    if mode == "reg":
        scratch.append(pltpu.VMEM((npad,), jnp.float32))

    def body(table_hbm, src_hbm, dst_hbm, zero_hbm, out_hbm,
             sidx, didx, rows, acc, sem_i, sem_s, *rest):
        cid = lax.axis_index("c")
        sid = lax.axis_index("s")
        wid = sid * NC + cid
        row0 = sid * rows_per_tile
        pltpu.sync_copy(zero_hbm.at[pl.ds(row0, rows_per_tile)],
                        acc.at[pl.ds(row0, rows_per_tile)])
        if mode == "count":
            for j in range(GRP // LANE):
                rows[pl.ds(j * LANE, LANE)] = jnp.ones((LANE,), jnp.float32)
        if mode == "reg":
            vtab = rest[0]
            pltpu.sync_copy(table_hbm, vtab)
        plsc.subcore_barrier()

        nf = (n_chunks + NW - 1 - wid) // NW

        sem_g = rest[:GB] if mode.startswith("dma") else ()

        def idx_start(i, buf):
            base = (wid + i * NW) * GB
            if mode != "count":
                pltpu.async_copy(src_hbm.at[pl.ds(base, GB)],
                                 sidx.at[buf], sem_i)
            pltpu.async_copy(dst_hbm.at[pl.ds(base, GB)], didx.at[buf], sem_i)

        def idx_wait(i, buf):
            base = (wid + i * NW) * GB
            if mode != "count":
                pltpu.make_async_copy(src_hbm.at[pl.ds(base, GB)],
                                      sidx.at[buf],
                                      sem_i).wait()
            pltpu.make_async_copy(dst_hbm.at[pl.ds(base, GB)], didx.at[buf],
                                  sem_i).wait()

        if mode == "dma_r1":
            def chunk_r1(i, carry):
                base = (wid + i * NW) * GB
                pltpu.sync_copy(src_hbm.at[pl.ds(base, GB)], sidx)
                pltpu.sync_copy(dst_hbm.at[pl.ds(base, GB)], didx.at[0])
                for g in range(GB):
                    pltpu.async_copy(table_hbm.at[sidx.at[g]], rows.at[g],
                                     sem_g[0]).wait()
                    pltpu.sync_copy(rows.at[g], acc.at[didx.at[0, g]],
                                    add=True)
                return carry
            lax.fori_loop(0, nf, chunk_r1, 0)
            plsc.subcore_barrier()
            pltpu.sync_copy(acc.at[pl.ds(row0, rows_per_tile)],
                            out_hbm.at[cid, pl.ds(row0, rows_per_tile)])
            return

        @pl.when(nf > 0)
        def _():
            idx_start(0, 0)

        def process(i, cur):
            idx_wait(i, cur)

            @pl.when(i + 1 < nf)
            def _():
                idx_start(i + 1, 1 - cur)

            if mode == "dma":
                for g in range(GB):
                    pltpu.async_copy(
                        table_hbm.at[sidx.at[cur, g]],
                        rows.at[g], sem_g[g])
                for g in range(GB):
                    pltpu.make_async_copy(
                        table_hbm.at[sidx.at[cur, g]],
                        rows.at[g], sem_g[g]).wait()
                    pltpu.async_copy(rows.at[g], acc.at[didx.at[cur, g]],
                                     sem_s, add=True)
            elif mode == "dma_sync":
                for g in range(GB):
                    pltpu.async_copy(
                        table_hbm.at[sidx.at[cur, g]],
                        rows.at[g], sem_g[0]).wait()
                    pltpu.async_copy(rows.at[g], acc.at[didx.at[cur, g]],
                                     sem_s, add=True)
            elif mode == "reg":
                vtab = rest[0]
                lanes = lax.iota(jnp.int32, LANE)
                for g in range(GB):
                    gcur = jnp.full((LANE,), cur, jnp.int32)
                    grow = jnp.full((LANE,), g, jnp.int32)
                    for j in range(GRP // LANE):
                        idxv = plsc.load_gather(sidx, [gcur, grow,
                                                       j * LANE + lanes])
                        rows[g, pl.ds(j * LANE, LANE)] = \
                            plsc.load_gather(vtab, [idxv])
                    pltpu.async_copy(rows.at[g], acc.at[didx.at[cur, g]],
                                     sem_s, add=True)
            else:
                for g in range(GB):
                    pltpu.async_copy(rows, acc.at[didx.at[cur, g]],
                                     sem_s, add=True)
            for g in range(GB):
                src_ref = rows if mode == "count" else rows.at[g]
                pltpu.make_async_copy(src_ref, acc.at[didx.at[cur, g]],
                                      sem_s).wait()

        def pair(p, carry):
            process(2 * p, 0)
            process(2 * p + 1, 1)
            return carry

        lax.fori_loop(0, nf // 2, pair, 0)

        @pl.when(lax.rem(nf, 2) == 1)
        def _():
            process(nf - 1, 0)
        plsc.subcore_barrier()
        pltpu.sync_copy(acc.at[pl.ds(row0, rows_per_tile)],
                        out_hbm.at[cid, pl.ds(row0, rows_per_tile)])

    return pl.kernel(body,
                     out_type=jax.ShapeDtypeStruct(out_shape, jnp.float32),
                     mesh=mesh, scratch_types=scratch,
                     compiler_params=pltpu.CompilerParams(
                         use_tc_tiling_on_sc=False,
                         needs_layout_passes=(mode != "reg")))


def _stage_deg(npad):
    grid = npad // ROWBLK
    dr = ROWBLK // 128

    def body(dp_ref, dinv_ref):
        dinv_ref[...] = lax.rsqrt(1.0 + dp_ref[0] + dp_ref[1])

    return pl.pallas_call(
        body,
        grid=(grid,),
        in_specs=[pl.BlockSpec((2, dr, 128), lambda i: (0, i, 0))],
        out_specs=pl.BlockSpec((dr, 128), lambda i: (i, 0)),
        out_shape=jax.ShapeDtypeStruct((npad // 128, 128), jnp.float32),
    )


def _stage_mid(npad, fp, f_hid):
    grid = npad // ROWBLK
    xr = ROWBLK * fp // 128

    def body(ap_ref, u_ref, m16_ref, m8_ref, w1_ref, b1_ref, w2_ref, v_ref):
        a = ap_ref[0] + ap_ref[1] + u_ref[...]
        t = jnp.dot(a, w1_ref[...], preferred_element_type=jnp.float32)
        h = jnp.maximum(t * m16_ref[...] + b1_ref[...], 0.0)
        v_ref[...] = jnp.dot(h, w2_ref[...],
                             preferred_element_type=jnp.float32) * m8_ref[...]

    return pl.pallas_call(
        body,
        grid=(grid,),
        in_specs=[pl.BlockSpec((2, xr, 128), lambda i: (0, i, 0)),
                  pl.BlockSpec((xr, 128), lambda i: (i, 0)),
                  pl.BlockSpec((xr, 16 * f_hid), lambda i: (i, 0)),
                  pl.BlockSpec((xr, 128), lambda i: (i, 0)),
                  pl.BlockSpec((16 * fp, 16 * f_hid), lambda i: (0, 0)),
                  pl.BlockSpec((1, 16 * f_hid), lambda i: (0, 0)),
                  pl.BlockSpec((16 * f_hid, 128), lambda i: (0, 0))],
        out_specs=pl.BlockSpec((xr, 128), lambda i: (i, 0)),
        out_shape=jax.ShapeDtypeStruct((npad * fp // 128, 128), jnp.float32),
    )


def _stage_out(npad):
    grid = npad // ROWBLK
    dr = ROWBLK // 128

    def body(ap_ref, v_ref, dinv_ref, b2_ref, o_ref):
        a = ap_ref[0] + ap_ref[1] + v_ref[...]
        o_ref[...] = a * dinv_ref[...] + b2_ref[0, 0]

    return pl.pallas_call(
        body,
        grid=(grid,),
        in_specs=[pl.BlockSpec((2, dr, 128), lambda i: (0, i, 0)),
                  pl.BlockSpec((dr, 128), lambda i: (i, 0)),
                  pl.BlockSpec((dr, 128), lambda i: (i, 0)),
                  pl.BlockSpec((1, 1), lambda i: (0, 0))],
        out_specs=pl.BlockSpec((dr, 128), lambda i: (i, 0)),
        out_shape=jax.ShapeDtypeStruct((npad // 128, 128), jnp.float32),
    )


def kernel(x, edge_index, W1, b1, W2, b2):
    n, f_in = x.shape
    f_hid = W1.shape[1]
    e = edge_index.shape[1]

    npad = _cdiv(n, ROWBLK) * ROWBLK
    if npad == n:
        npad += ROWBLK

    if e % CHW == 0:
        src = edge_index[0]
        dst = edge_index[1]
        epad = e
    else:
        epad = _cdiv(e, CHW) * CHW
        fill = jnp.full((epad - e,), n, jnp.int32)
        src = jnp.concatenate([edge_index[0], fill])
        dst = jnp.concatenate([edge_index[1], fill])
    n_chunks = epad // CHW
    src2 = src.reshape(-1, GRP)
    dst2 = dst.reshape(-1, GRP)

    fp = 8
    xp = jnp.zeros((npad, fp), jnp.float32).at[:n, :f_in].set(x)
    w1big = jnp.kron(jnp.eye(16, dtype=jnp.float32),
                     jnp.zeros((fp, f_hid), jnp.float32).at[:f_in].set(W1))
    w2big = jnp.kron(jnp.eye(16, dtype=jnp.float32),
                     W2 * jnp.ones((1, fp), jnp.float32))
    b1big = jnp.tile(b1, 16).reshape(1, 16 * f_hid)
    z1 = jnp.zeros((npad,), jnp.float32)
    z8 = jnp.zeros((npad, fp), jnp.float32)

    deg_p = _make_agg(npad, n_chunks, 1, "count")(z1, src2, dst2, z1)
    dinvf = _stage_deg(npad)(deg_p.reshape(NC, npad // 128, 128))
    dinv = dinvf.reshape(npad)
    uf = xp.reshape(npad * fp // 128, 128) * \
        jnp.repeat(dinv, fp).reshape(npad * fp // 128, 128)
    m16 = jnp.repeat(dinv, 16).reshape(npad // 16, 16 * f_hid)
    agg1_p = _make_agg(npad, n_chunks, fp, "dma")(uf.reshape(npad, fp),
                                                  src2, dst2, z8)
    v8f = _stage_mid(npad, fp, f_hid)(
        agg1_p.reshape(NC, npad * fp // 128, 128), uf, m16,
        jnp.repeat(dinv, fp).reshape(npad * fp // 128, 128),
        w1big, b1big, w2big)
    v1 = v8f.reshape(npad, fp)[:, 0]
    agg2_p = _make_agg(npad, n_chunks, 1, "reg")(v1, src2, dst2, z1)
    outf = _stage_out(npad)(agg2_p.reshape(NC, npad // 128, 128),
                            v1.reshape(npad // 128, 128), dinvf,
                            b2.reshape(1, 1))
    return outf.reshape(npad, 1)[:n]


# speedup vs baseline: 155.5118x; 1.0009x over previous
import functools

import jax
import jax.numpy as jnp
from jax import lax
from jax.experimental import pallas as pl
from jax.experimental.pallas import tpu as pltpu
from jax.experimental.pallas import tpu_sc as plsc

NC = 2
NS = 16
NW = NC * NS
LANE = 16
GRP = 128
GB = 8
CHW = GB * GRP
ROWBLK = 2048


def _cdiv(a, b):
    return (a + b - 1) // b


@functools.lru_cache(maxsize=None)
def _make_agg(npad, n_chunks, d, mode):
    mesh = plsc.VectorSubcoreMesh(core_axis_name="c", subcore_axis_name="s",
                                  num_cores=NC, num_subcores=NS)
    rows_per_tile = npad // NS
    vec2 = d > 1
    acc_shape = (npad, d) if vec2 else (npad,)
    out_shape = (NC, npad, d) if vec2 else (NC, npad)
    rows_shape = ((GB, GRP, d) if vec2 else (GB, GRP)) if mode != "count" \
        else (GRP,)

    scratch = [
        pltpu.VMEM((2, GB, GRP), jnp.int32),
        pltpu.VMEM((2, GB, GRP), jnp.int32),
        pltpu.VMEM(rows_shape, jnp.float32),
        pltpu.VMEM_SHARED(acc_shape, jnp.float32),
        pltpu.SemaphoreType.DMA,
        pltpu.SemaphoreType.DMA,
    ]
    if mode == "dma":
        scratch.extend([pltpu.SemaphoreType.DMA] * GB)
    if mode == "reg":
        scratch.append(pltpu.VMEM((npad,), jnp.float32))

    def body(table_hbm, src_hbm, dst_hbm, zero_hbm, out_hbm,
             sidx, didx, rows, acc, sem_i, sem_s, *rest):
        cid = lax.axis_index("c")
        sid = lax.axis_index("s")
        wid = sid * NC + cid
        row0 = sid * rows_per_tile
        pltpu.sync_copy(zero_hbm.at[pl.ds(row0, rows_per_tile)],
                        acc.at[pl.ds(row0, rows_per_tile)])
        if mode == "count":
            for j in range(GRP // LANE):
                rows[pl.ds(j * LANE, LANE)] = jnp.ones((LANE,), jnp.float32)
        if mode == "reg":
            vtab = rest[0]
            pltpu.sync_copy(table_hbm, vtab)
        plsc.subcore_barrier()

        nf = (n_chunks + NW - 1 - wid) // NW

        sem_g = rest[:GB] if mode == "dma" else ()

        def idx_start(i, buf):
            base = (wid + i * NW) * GB
            if mode != "count":
                pltpu.async_copy(src_hbm.at[pl.ds(base, GB)],
                                 sidx.at[buf], sem_i)
            pltpu.async_copy(dst_hbm.at[pl.ds(base, GB)], didx.at[buf], sem_i)

        def idx_wait(i, buf):
            base = (wid + i * NW) * GB
            if mode != "count":
                pltpu.make_async_copy(src_hbm.at[pl.ds(base, GB)],
                                      sidx.at[buf],
                                      sem_i).wait()
            pltpu.make_async_copy(dst_hbm.at[pl.ds(base, GB)], didx.at[buf],
                                  sem_i).wait()

        @pl.when(nf > 0)
        def _():
            idx_start(0, 0)

        def process(i, cur):
            idx_wait(i, cur)

            @pl.when(i + 1 < nf)
            def _():
                idx_start(i + 1, 1 - cur)

            if mode == "dma":
                for g in range(GB):
                    pltpu.async_copy(
                        table_hbm.at[sidx.at[cur, g]],
                        rows.at[g], sem_g[g])
                for g in range(GB):
                    pltpu.make_async_copy(
                        table_hbm.at[sidx.at[cur, g]],
                        rows.at[g], sem_g[g]).wait()
                    pltpu.async_copy(rows.at[g], acc.at[didx.at[cur, g]],
                                     sem_s, add=True)
            elif mode == "reg":
                vtab = rest[0]
                lanes = lax.iota(jnp.int32, LANE)
                for g in range(GB):
                    gcur = jnp.full((LANE,), cur, jnp.int32)
                    grow = jnp.full((LANE,), g, jnp.int32)
                    for j in range(GRP // LANE):
                        idxv = plsc.load_gather(sidx, [gcur, grow,
                                                       j * LANE + lanes])
                        rows[g, pl.ds(j * LANE, LANE)] = \
                            plsc.load_gather(vtab, [idxv])
                    pltpu.async_copy(rows.at[g], acc.at[didx.at[cur, g]],
                                     sem_s, add=True)
            else:
                for g in range(GB):
                    pltpu.async_copy(rows, acc.at[didx.at[cur, g]],
                                     sem_s, add=True)
            for g in range(GB):
                src_ref = rows if mode == "count" else rows.at[g]
                pltpu.make_async_copy(src_ref, acc.at[didx.at[cur, g]],
                                      sem_s).wait()

        def pair(p, carry):
            process(2 * p, 0)
            process(2 * p + 1, 1)
            return carry

        lax.fori_loop(0, nf // 2, pair, 0)

        @pl.when(lax.rem(nf, 2) == 1)
        def _():
            process(nf - 1, 0)
        plsc.subcore_barrier()
        pltpu.sync_copy(acc.at[pl.ds(row0, rows_per_tile)],
                        out_hbm.at[cid, pl.ds(row0, rows_per_tile)])

    return pl.kernel(body,
                     out_type=jax.ShapeDtypeStruct(out_shape, jnp.float32),
                     mesh=mesh, scratch_types=scratch,
                     compiler_params=pltpu.CompilerParams(
                         use_tc_tiling_on_sc=False,
                         needs_layout_passes=(mode != "reg")))


def _stage_deg(npad):
    grid = npad // ROWBLK
    dr = ROWBLK // 128

    def body(dp_ref, dinv_ref):
        dinv_ref[...] = lax.rsqrt(1.0 + dp_ref[0] + dp_ref[1])

    return pl.pallas_call(
        body,
        grid=(grid,),
        in_specs=[pl.BlockSpec((2, dr, 128), lambda i: (0, i, 0))],
        out_specs=pl.BlockSpec((dr, 128), lambda i: (i, 0)),
        out_shape=jax.ShapeDtypeStruct((npad // 128, 128), jnp.float32),
    )


def _stage_mid(npad, fp, f_hid):
    grid = npad // ROWBLK
    xr = ROWBLK * fp // 128

    def body(ap_ref, u_ref, m16_ref, m8_ref, w1_ref, b1_ref, w2_ref, v_ref):
        a = ap_ref[0] + ap_ref[1] + u_ref[...]
        t = jnp.dot(a, w1_ref[...], preferred_element_type=jnp.float32)
        h = jnp.maximum(t * m16_ref[...] + b1_ref[...], 0.0)
        v_ref[...] = jnp.dot(h, w2_ref[...],
                             preferred_element_type=jnp.float32) * m8_ref[...]

    return pl.pallas_call(
        body,
        grid=(grid,),
        in_specs=[pl.BlockSpec((2, xr, 128), lambda i: (0, i, 0)),
                  pl.BlockSpec((xr, 128), lambda i: (i, 0)),
                  pl.BlockSpec((xr, 16 * f_hid), lambda i: (i, 0)),
                  pl.BlockSpec((xr, 128), lambda i: (i, 0)),
                  pl.BlockSpec((16 * fp, 16 * f_hid), lambda i: (0, 0)),
                  pl.BlockSpec((1, 16 * f_hid), lambda i: (0, 0)),
                  pl.BlockSpec((16 * f_hid, 128), lambda i: (0, 0))],
        out_specs=pl.BlockSpec((xr, 128), lambda i: (i, 0)),
        out_shape=jax.ShapeDtypeStruct((npad * fp // 128, 128), jnp.float32),
    )


def _stage_out(npad):
    grid = npad // ROWBLK
    dr = ROWBLK // 128

    def body(ap_ref, v_ref, dinv_ref, b2_ref, o_ref):
        a = ap_ref[0] + ap_ref[1] + v_ref[...]
        o_ref[...] = a * dinv_ref[...] + b2_ref[0, 0]

    return pl.pallas_call(
        body,
        grid=(grid,),
        in_specs=[pl.BlockSpec((2, dr, 128), lambda i: (0, i, 0)),
                  pl.BlockSpec((dr, 128), lambda i: (i, 0)),
                  pl.BlockSpec((dr, 128), lambda i: (i, 0)),
                  pl.BlockSpec((1, 1), lambda i: (0, 0))],
        out_specs=pl.BlockSpec((dr, 128), lambda i: (i, 0)),
        out_shape=jax.ShapeDtypeStruct((npad // 128, 128), jnp.float32),
    )


def kernel(x, edge_index, W1, b1, W2, b2):
    n, f_in = x.shape
    f_hid = W1.shape[1]
    e = edge_index.shape[1]

    npad = _cdiv(n, ROWBLK) * ROWBLK
    if npad == n:
        npad += ROWBLK

    if e % CHW == 0:
        src = edge_index[0]
        dst = edge_index[1]
        epad = e
    else:
        epad = _cdiv(e, CHW) * CHW
        fill = jnp.full((epad - e,), n, jnp.int32)
        src = jnp.concatenate([edge_index[0], fill])
        dst = jnp.concatenate([edge_index[1], fill])
    n_chunks = epad // CHW
    src2 = src.reshape(-1, GRP)
    dst2 = dst.reshape(-1, GRP)

    fp = 8
    xp = jnp.zeros((npad, fp), jnp.float32).at[:n, :f_in].set(x)
    w1big = jnp.kron(jnp.eye(16, dtype=jnp.float32),
                     jnp.zeros((fp, f_hid), jnp.float32).at[:f_in].set(W1))
    w2big = jnp.kron(jnp.eye(16, dtype=jnp.float32),
                     W2 * jnp.ones((1, fp), jnp.float32))
    b1big = jnp.tile(b1, 16).reshape(1, 16 * f_hid)
    z1 = jnp.zeros((npad,), jnp.float32)
    z8 = jnp.zeros((npad, fp), jnp.float32)

    deg_p = _make_agg(npad, n_chunks, 1, "count")(z1, src2, dst2, z1)
    dinvf = _stage_deg(npad)(deg_p.reshape(NC, npad // 128, 128))
    dinv = dinvf.reshape(npad)
    uf = xp.reshape(npad * fp // 128, 128) * \
        jnp.repeat(dinv, fp).reshape(npad * fp // 128, 128)
    m16 = jnp.repeat(dinv, 16).reshape(npad // 16, 16 * f_hid)
    agg1_p = _make_agg(npad, n_chunks, fp, "dma")(uf.reshape(npad, fp),
                                                  src2, dst2, z8)
    v8f = _stage_mid(npad, fp, f_hid)(
        agg1_p.reshape(NC, npad * fp // 128, 128), uf, m16,
        jnp.repeat(dinv, fp).reshape(npad * fp // 128, 128),
        w1big, b1big, w2big)
    v1 = v8f.reshape(npad, fp)[:, 0]
    agg2_p = _make_agg(npad, n_chunks, 1, "reg")(v1, src2, dst2, z1)
    outf = _stage_out(npad)(agg2_p.reshape(NC, npad // 128, 128),
                            v1.reshape(npad // 128, 128), dinvf,
                            b2.reshape(1, 1))
    return outf.reshape(npad, 1)[:n]


# speedup vs baseline: 161.0094x; 1.0354x over previous
import functools

import jax
import jax.numpy as jnp
from jax import lax
from jax.experimental import pallas as pl
from jax.experimental.pallas import tpu as pltpu
from jax.experimental.pallas import tpu_sc as plsc

NC = 2
NS = 16
NW = NC * NS
LANE = 16
GRP = 128
GB = 8
CHW = GB * GRP
ROWBLK = 2048


def _cdiv(a, b):
    return (a + b - 1) // b


@functools.lru_cache(maxsize=None)
def _make_agg(npad, n_chunks, d, mode):
    mesh = plsc.VectorSubcoreMesh(core_axis_name="c", subcore_axis_name="s",
                                  num_cores=NC, num_subcores=NS)
    rows_per_tile = npad // NS
    vec2 = d > 1
    acc_shape = (npad, d) if vec2 else (npad,)
    out_shape = (NC, npad, d) if vec2 else (NC, npad)
    rows_shape = ((GB, GRP, d) if vec2 else (GB, GRP)) if mode != "count" \
        else (GRP,)

    scratch = [
        pltpu.VMEM((2, GB, GRP), jnp.int32),
        pltpu.VMEM((2, GB, GRP), jnp.int32),
        pltpu.VMEM(rows_shape, jnp.float32),
        pltpu.VMEM_SHARED(acc_shape, jnp.float32),
        pltpu.SemaphoreType.DMA,
        pltpu.SemaphoreType.DMA,
    ]
    if mode == "dma":
        scratch.extend([pltpu.SemaphoreType.DMA] * GB)
    if mode == "reg":
        scratch.append(pltpu.VMEM((npad,), jnp.float32))

    def body(table_hbm, src_hbm, dst_hbm, zero_hbm, out_hbm,
             sidx, didx, rows, acc, sem_i, sem_s, *rest):
        cid = lax.axis_index("c")
        sid = lax.axis_index("s")
        wid = sid * NC + cid
        row0 = sid * rows_per_tile
        if mode == "count":
            pltpu.sync_copy(zero_hbm.at[pl.ds(row0, rows_per_tile)],
                            acc.at[pl.ds(row0, rows_per_tile)])
        else:
            @pl.when(cid == 0)
            def _():
                pltpu.sync_copy(table_hbm.at[pl.ds(row0, rows_per_tile)],
                                acc.at[pl.ds(row0, rows_per_tile)])

            @pl.when(cid != 0)
            def _():
                pltpu.sync_copy(zero_hbm.at[pl.ds(row0, rows_per_tile)],
                                acc.at[pl.ds(row0, rows_per_tile)])
        if mode == "count":
            for j in range(GRP // LANE):
                rows[pl.ds(j * LANE, LANE)] = jnp.ones((LANE,), jnp.float32)
        if mode == "reg":
            vtab = rest[0]
            pltpu.sync_copy(table_hbm, vtab)
        plsc.subcore_barrier()

        nf = (n_chunks + NW - 1 - wid) // NW

        sem_g = rest[:GB] if mode == "dma" else ()

        def idx_start(i, buf):
            base = (wid + i * NW) * GB
            if mode != "count":
                pltpu.async_copy(src_hbm.at[pl.ds(base, GB)],
                                 sidx.at[buf], sem_i)
            pltpu.async_copy(dst_hbm.at[pl.ds(base, GB)], didx.at[buf], sem_i)

        def idx_wait(i, buf):
            base = (wid + i * NW) * GB
            if mode != "count":
                pltpu.make_async_copy(src_hbm.at[pl.ds(base, GB)],
                                      sidx.at[buf],
                                      sem_i).wait()
            pltpu.make_async_copy(dst_hbm.at[pl.ds(base, GB)], didx.at[buf],
                                  sem_i).wait()

        @pl.when(nf > 0)
        def _():
            idx_start(0, 0)

        def process(i, cur):
            idx_wait(i, cur)

            @pl.when(i + 1 < nf)
            def _():
                idx_start(i + 1, 1 - cur)

            if mode == "dma":
                for g in range(GB):
                    pltpu.async_copy(
                        table_hbm.at[sidx.at[cur, g]],
                        rows.at[g], sem_g[g])
                for g in range(GB):
                    pltpu.make_async_copy(
                        table_hbm.at[sidx.at[cur, g]],
                        rows.at[g], sem_g[g]).wait()
                    pltpu.async_copy(rows.at[g], acc.at[didx.at[cur, g]],
                                     sem_s, add=True)
            elif mode == "reg":
                vtab = rest[0]
                lanes = lax.iota(jnp.int32, LANE)
                for g in range(GB):
                    gcur = jnp.full((LANE,), cur, jnp.int32)
                    grow = jnp.full((LANE,), g, jnp.int32)
                    for j in range(GRP // LANE):
                        idxv = plsc.load_gather(sidx, [gcur, grow,
                                                       j * LANE + lanes])
                        rows[g, pl.ds(j * LANE, LANE)] = \
                            plsc.load_gather(vtab, [idxv])
                    pltpu.async_copy(rows.at[g], acc.at[didx.at[cur, g]],
                                     sem_s, add=True)
            else:
                for g in range(GB):
                    pltpu.async_copy(rows, acc.at[didx.at[cur, g]],
                                     sem_s, add=True)
            for g in range(GB):
                src_ref = rows if mode == "count" else rows.at[g]
                pltpu.make_async_copy(src_ref, acc.at[didx.at[cur, g]],
                                      sem_s).wait()

        def pair(p, carry):
            process(2 * p, 0)
            process(2 * p + 1, 1)
            return carry

        lax.fori_loop(0, nf // 2, pair, 0)

        @pl.when(lax.rem(nf, 2) == 1)
        def _():
            process(nf - 1, 0)
        plsc.subcore_barrier()
        pltpu.sync_copy(acc.at[pl.ds(row0, rows_per_tile)],
                        out_hbm.at[cid, pl.ds(row0, rows_per_tile)])

    return pl.kernel(body,
                     out_type=jax.ShapeDtypeStruct(out_shape, jnp.float32),
                     mesh=mesh, scratch_types=scratch,
                     compiler_params=pltpu.CompilerParams(
                         use_tc_tiling_on_sc=False,
                         needs_layout_passes=(mode != "reg")))


def _stage_deg(npad):
    grid = npad // ROWBLK
    dr = ROWBLK // 128

    def body(dp_ref, dinv_ref):
        dinv_ref[...] = lax.rsqrt(1.0 + dp_ref[0] + dp_ref[1])

    return pl.pallas_call(
        body,
        grid=(grid,),
        in_specs=[pl.BlockSpec((2, dr, 128), lambda i: (0, i, 0))],
        out_specs=pl.BlockSpec((dr, 128), lambda i: (i, 0)),
        out_shape=jax.ShapeDtypeStruct((npad // 128, 128), jnp.float32),
    )


def _stage_mid(npad, fp, f_hid):
    grid = npad // ROWBLK
    xr = ROWBLK * fp // 128

    def body(ap_ref, m16_ref, m8_ref, w1_ref, b1_ref, w2_ref, v_ref):
        a = ap_ref[0] + ap_ref[1]
        t = jnp.dot(a, w1_ref[...], preferred_element_type=jnp.float32)
        h = jnp.maximum(t * m16_ref[...] + b1_ref[...], 0.0)
        v_ref[...] = jnp.dot(h, w2_ref[...],
                             preferred_element_type=jnp.float32) * m8_ref[...]

    return pl.pallas_call(
        body,
        grid=(grid,),
        in_specs=[pl.BlockSpec((2, xr, 128), lambda i: (0, i, 0)),
                  pl.BlockSpec((xr, 16 * f_hid), lambda i: (i, 0)),
                  pl.BlockSpec((xr, 128), lambda i: (i, 0)),
                  pl.BlockSpec((16 * fp, 16 * f_hid), lambda i: (0, 0)),
                  pl.BlockSpec((1, 16 * f_hid), lambda i: (0, 0)),
                  pl.BlockSpec((16 * f_hid, 128), lambda i: (0, 0))],
        out_specs=pl.BlockSpec((xr, 128), lambda i: (i, 0)),
        out_shape=jax.ShapeDtypeStruct((npad * fp // 128, 128), jnp.float32),
    )


def _stage_out(npad):
    grid = npad // ROWBLK
    dr = ROWBLK // 128

    def body(ap_ref, dinv_ref, b2_ref, o_ref):
        a = ap_ref[0] + ap_ref[1]
        o_ref[...] = a * dinv_ref[...] + b2_ref[0, 0]

    return pl.pallas_call(
        body,
        grid=(grid,),
        in_specs=[pl.BlockSpec((2, dr, 128), lambda i: (0, i, 0)),
                  pl.BlockSpec((dr, 128), lambda i: (i, 0)),
                  pl.BlockSpec((1, 1), lambda i: (0, 0))],
        out_specs=pl.BlockSpec((dr, 128), lambda i: (i, 0)),
        out_shape=jax.ShapeDtypeStruct((npad // 128, 128), jnp.float32),
    )


def kernel(x, edge_index, W1, b1, W2, b2):
    n, f_in = x.shape
    f_hid = W1.shape[1]
    e = edge_index.shape[1]

    npad = _cdiv(n, ROWBLK) * ROWBLK
    if npad == n:
        npad += ROWBLK

    if e % CHW == 0:
        src = edge_index[0]
        dst = edge_index[1]
        epad = e
    else:
        epad = _cdiv(e, CHW) * CHW
        fill = jnp.full((epad - e,), n, jnp.int32)
        src = jnp.concatenate([edge_index[0], fill])
        dst = jnp.concatenate([edge_index[1], fill])
    n_chunks = epad // CHW
    src2 = src.reshape(-1, GRP)
    dst2 = dst.reshape(-1, GRP)

    fp = 8
    xp = jnp.zeros((npad, fp), jnp.float32).at[:n, :f_in].set(x)
    w1big = jnp.kron(jnp.eye(16, dtype=jnp.float32),
                     jnp.zeros((fp, f_hid), jnp.float32).at[:f_in].set(W1))
    w2big = jnp.kron(jnp.eye(16, dtype=jnp.float32),
                     W2 * jnp.ones((1, fp), jnp.float32))
    b1big = jnp.tile(b1, 16).reshape(1, 16 * f_hid)
    z1 = jnp.zeros((npad,), jnp.float32)
    z8 = jnp.zeros((npad, fp), jnp.float32)

    deg_p = _make_agg(npad, n_chunks, 1, "count")(z1, src2, dst2, z1)
    dinvf = _stage_deg(npad)(deg_p.reshape(NC, npad // 128, 128))
    dinv = dinvf.reshape(npad)
    u = xp * dinv[:, None]
    m16 = jnp.broadcast_to(dinv.reshape(npad // 16, 16, 1),
                           (npad // 16, 16, f_hid)) \
        .reshape(npad // 16, 16 * f_hid)
    m8 = jnp.broadcast_to(dinv.reshape(npad // 16, 16, 1),
                          (npad // 16, 16, fp)) \
        .reshape(npad * fp // 128, 128)
    agg1_p = _make_agg(npad, n_chunks, fp, "dma")(u, src2, dst2, z8)
    v8f = _stage_mid(npad, fp, f_hid)(
        agg1_p.reshape(NC, npad * fp // 128, 128), m16, m8,
        w1big, b1big, w2big)
    v1 = v8f.reshape(npad, fp)[:, 0]
    agg2_p = _make_agg(npad, n_chunks, 1, "reg")(v1, src2, dst2, z1)
    outf = _stage_out(npad)(agg2_p.reshape(NC, npad // 128, 128), dinvf,
                            b2.reshape(1, 1))
    return outf.reshape(npad, 1)[:n]
